# baseline (speedup 1.0000x reference)
import functools

import jax
import jax.numpy as jnp
from jax import lax
from jax.experimental import pallas as pl
from jax.experimental.pallas import tpu as pltpu
from jax.experimental.pallas import tpu_sc as plsc

_N = 50000
_E = 800000
_D_IN = 176
_H = 64
_HH = 32
_L = 4

_NS = 25000
_EPADH = 409600
_ROWS = 2 * _EPADH // 128
_NSUB = 16
_RPT = _EPADH // 128 // _NSUB
_CROWS = 4
_CHUNK = _CROWS * 128
_NCHUNK = _RPT // _CROWS
_AGGR = 26000
_STRIPE = 1568
_ZROWS = 392
_ZREP = _STRIPE // _ZROWS

_BR = 1000
_NBLK = _N // _BR
_XBLK = _NS // _BR
_HIBLK = _AGGR // _BR


def _sc_layer_body(x_part, hlo_hbm, hhi_hbm, src_hbm, dst_hbm, eblo_hbm,
                   ebhi_hbm, out_hbm,
                   gbuf0, gbuf1, eb0, eb1, sv0, sv1, dv0, dv1, aggsp,
                   gsem0, gsem1, ssem0, ssem1):
    c = lax.axis_index("c")
    s = lax.axis_index("s")

    zv = jnp.zeros((_HH,), jnp.float32)

    def zrow(i, carry):
        gbuf0[i] = zv
        return carry

    lax.fori_loop(0, _ZROWS, zrow, 0)
    base = s * _STRIPE
    for t in range(_ZREP):
        pltpu.sync_copy(gbuf0.at[pl.ds(0, _ZROWS)],
                        aggsp.at[pl.ds(base + t * _ZROWS, _ZROWS)])
    plsc.subcore_barrier()

    row0 = x_part * _RPT * _NSUB + s * _RPT
    ebase = x_part * _EPADH

    set0 = (gbuf0, eb0, sv0, dv0, gsem0, ssem0)
    set1 = (gbuf1, eb1, sv1, dv1, gsem1, ssem1)

    def prefetch(g, bufs):
        gbuf, ebuf, sv, dv, gsem, _ = bufs
        rb = row0 + g * _CROWS
        pltpu.sync_copy(src_hbm.at[pl.ds(rb, _CROWS)], sv)
        pltpu.sync_copy(dst_hbm.at[pl.ds(rb, _CROWS)], dv)
        erb = ebase + (s * _RPT + g * _CROWS) * 128

        @pl.when(c == 0)
        def _():
            pltpu.sync_copy(eblo_hbm.at[pl.ds(erb, _CHUNK)], ebuf)
            for j in range(_CROWS):
                pltpu.async_copy(hlo_hbm.at[sv.at[j]],
                                 gbuf.at[pl.ds(j * 128, 128)], gsem)

        @pl.when(c == 1)
        def _():
            pltpu.sync_copy(ebhi_hbm.at[pl.ds(erb, _CHUNK)], ebuf)
            for j in range(_CROWS):
                pltpu.async_copy(hhi_hbm.at[sv.at[j]],
                                 gbuf.at[pl.ds(j * 128, 128)], gsem)

    def wait_n(sem, gbuf):
        for j in range(_CROWS):
            pltpu.make_async_copy(hlo_hbm.at[pl.ds(0, 128)],
                                  gbuf.at[pl.ds(0, 128)], sem).wait()

    def process(bufs):
        gbuf, ebuf, sv, dv, gsem, ssem = bufs
        wait_n(gsem, gbuf)

        def edge8(t, cr):
            e0 = t * 8
            for i in range(8):
                e = e0 + i
                gbuf[e] = jnp.maximum(gbuf[e] + ebuf[e], 0.0)
            return cr

        lax.fori_loop(0, _CHUNK // 8, edge8, 0)
        for j in range(_CROWS):
            pltpu.async_copy(gbuf.at[pl.ds(j * 128, 128)],
                             aggsp.at[dv.at[j]], ssem, add=True)

    def drain(bufs):
        gbuf, _, _, _, _, ssem = bufs
        wait_n(ssem, gbuf)

    prefetch(0, set0)
    prefetch(1, set1)
    process(set0)

    def pairbody(i, cr):
        t = 1 + 2 * i
        drain(set0)
        prefetch(t + 1, set0)
        process(set1)
        drain(set1)
        prefetch(t + 2, set1)
        process(set0)
        return cr

    lax.fori_loop(0, (_NCHUNK - 2) // 2, pairbody, 0)
    process(set1)
    drain(set0)
    drain(set1)
    plsc.subcore_barrier()

    obase = c * _AGGR + s * _STRIPE
    for t in range(_ZREP):
        pltpu.sync_copy(aggsp.at[pl.ds(base + t * _ZROWS, _ZROWS)],
                        gbuf0.at[pl.ds(0, _ZROWS)])
        pltpu.sync_copy(gbuf0.at[pl.ds(0, _ZROWS)],
                        out_hbm.at[pl.ds(obase + t * _ZROWS, _ZROWS)])


def _make_sc_layer(x_part):
    return pl.kernel(
        functools.partial(_sc_layer_body, x_part),
        out_type=jax.ShapeDtypeStruct((2 * _AGGR, _HH), jnp.float32),
        mesh=plsc.VectorSubcoreMesh(core_axis_name="c", subcore_axis_name="s",
                                    num_cores=2, num_subcores=_NSUB),
        scratch_types=[
            pltpu.VMEM((_CHUNK, _HH), jnp.float32),
            pltpu.VMEM((_CHUNK, _HH), jnp.float32),
            pltpu.VMEM((_CHUNK, _HH), jnp.float32),
            pltpu.VMEM((_CHUNK, _HH), jnp.float32),
            pltpu.VMEM((_CROWS, 128), jnp.int32),
            pltpu.VMEM((_CROWS, 128), jnp.int32),
            pltpu.VMEM((_CROWS, 128), jnp.int32),
            pltpu.VMEM((_CROWS, 128), jnp.int32),
            pltpu.VMEM_SHARED((_AGGR, _HH), jnp.float32),
            pltpu.SemaphoreType.DMA,
            pltpu.SemaphoreType.DMA,
            pltpu.SemaphoreType.DMA,
            pltpu.SemaphoreType.DMA,
        ],
        compiler_params=pltpu.CompilerParams(use_tc_tiling_on_sc=False),
    )


_sc_layer_x0 = _make_sc_layer(0)
_sc_layer_x1 = _make_sc_layer(1)


_BRE = 8192


def _tc_ebias_body(ea_ref, we_ref, be_ref, lo_ref, hi_ref):
    h = be_ref[...]
    for k in range(4):
        h = h + ea_ref[:, k:k + 1] * we_ref[0, k:k + 1, :]
    lo_ref[...] = h[:, :_HH]
    hi_ref[...] = h[:, _HH:]


_tc_ebias = pl.pallas_call(
    _tc_ebias_body,
    grid=(2 * _EPADH // _BRE,),
    in_specs=[
        pl.BlockSpec((_BRE, 4), lambda i: (i, 0)),
        pl.BlockSpec((1, 4, _H), lambda i: (0, 0, 0)),
        pl.BlockSpec((1, _H), lambda i: (0, 0)),
    ],
    out_specs=[pl.BlockSpec((_BRE, _HH), lambda i: (i, 0)),
               pl.BlockSpec((_BRE, _HH), lambda i: (i, 0))],
    out_shape=[jax.ShapeDtypeStruct((2 * _EPADH, _HH), jnp.float32),
               jax.ShapeDtypeStruct((2 * _EPADH, _HH), jnp.float32)],
)


def _tc_in_body(x_ref, w_ref, b_ref, lo_ref, hi_ref):
    h = jnp.dot(x_ref[...], w_ref[...],
                preferred_element_type=jnp.float32) + b_ref[...]
    lo_ref[...] = h[:, :_HH]
    hi_ref[...] = h[:, _HH:]


_tc_in = pl.pallas_call(
    _tc_in_body,
    grid=(_NBLK,),
    in_specs=[
        pl.BlockSpec((_BR, _D_IN), lambda i: (i, 0)),
        pl.BlockSpec((_D_IN, _H), lambda i: (0, 0)),
        pl.BlockSpec((1, _H), lambda i: (0, 0)),
    ],
    out_specs=[pl.BlockSpec((_BR, _HH), lambda i: (i, 0)),
               pl.BlockSpec((_BR, _HH), lambda i: (i, 0))],
    out_shape=[jax.ShapeDtypeStruct((_N, _HH), jnp.float32),
               jax.ShapeDtypeStruct((_N, _HH), jnp.float32)],
)


def _node_update(hlo_ref, hhi_ref, a0lo_ref, a0hi_ref, a1lo_ref, a1hi_ref,
                 w1_ref, b1_ref, w2_ref, b2_ref, gm_ref, bt_ref):
    in0 = pl.program_id(0) < _XBLK
    alo = jnp.where(in0, a0lo_ref[...], a1lo_ref[...])
    ahi = jnp.where(in0, a0hi_ref[...], a1hi_ref[...])
    z = jnp.concatenate(
        [hlo_ref[...] + alo, hhi_ref[...] + ahi], axis=1)
    t = jnp.maximum(jnp.dot(z, w1_ref[...],
                            preferred_element_type=jnp.float32)
                    + b1_ref[...], 0.0)
    z2 = jnp.dot(t, w2_ref[...],
                 preferred_element_type=jnp.float32) + b2_ref[...]
    mu = jnp.mean(z2, axis=1, keepdims=True)
    var = jnp.mean((z2 - mu) ** 2, axis=1, keepdims=True)
    zn = (z2 - mu) * lax.rsqrt(var + 1e-5)
    return jnp.maximum(zn * gm_ref[...] + bt_ref[...], 0.0)


def _tc_up_body(*refs):
    hn = _node_update(*refs[:-2])
    refs[-2][...] = hn[:, :_HH]
    refs[-1][...] = hn[:, _HH:]


def _tc_up_last_body(*refs):
    hn = _node_update(*refs[:-1])
    out_ref = refs[-1]

    @pl.when(pl.program_id(0) == 0)
    def _():
        out_ref[...] = jnp.zeros_like(out_ref)

    out_ref[...] += jnp.sum(hn, axis=0, keepdims=True)


_up_in_specs = [
    pl.BlockSpec((_BR, _HH), lambda i: (i, 0)),
    pl.BlockSpec((_BR, _HH), lambda i: (i, 0)),
    pl.BlockSpec((_BR, _HH), lambda i: (jnp.minimum(i, _XBLK - 1), 0)),
    pl.BlockSpec((_BR, _HH),
                 lambda i: (_HIBLK + jnp.minimum(i, _XBLK - 1), 0)),
    pl.BlockSpec((_BR, _HH), lambda i: (jnp.maximum(i - _XBLK, 0), 0)),
    pl.BlockSpec((_BR, _HH),
                 lambda i: (_HIBLK + jnp.maximum(i - _XBLK, 0), 0)),
    pl.BlockSpec((_H, 2 * _H), lambda i: (0, 0)),
    pl.BlockSpec((1, 2 * _H), lambda i: (0, 0)),
    pl.BlockSpec((2 * _H, _H), lambda i: (0, 0)),
    pl.BlockSpec((1, _H), lambda i: (0, 0)),
    pl.BlockSpec((1, _H), lambda i: (0, 0)),
    pl.BlockSpec((1, _H), lambda i: (0, 0)),
]

_tc_up = pl.pallas_call(
    _tc_up_body,
    grid=(_NBLK,),
    in_specs=_up_in_specs,
    out_specs=[pl.BlockSpec((_BR, _HH), lambda i: (i, 0)),
               pl.BlockSpec((_BR, _HH), lambda i: (i, 0))],
    out_shape=[jax.ShapeDtypeStruct((_N, _HH), jnp.float32),
               jax.ShapeDtypeStruct((_N, _HH), jnp.float32)],
)

_tc_up_last = pl.pallas_call(
    _tc_up_last_body,
    grid=(_NBLK,),
    in_specs=_up_in_specs,
    out_specs=pl.BlockSpec((1, _H), lambda i: (0, 0)),
    out_shape=jax.ShapeDtypeStruct((1, _H), jnp.float32),
)


def _tc_head_body(g_ref, wr1_ref, br1_ref, wr2_ref, br2_ref, out_ref):
    t = jnp.maximum(jnp.dot(g_ref[...], wr1_ref[...],
                            preferred_element_type=jnp.float32)
                    + br1_ref[...], 0.0)
    out_ref[...] = jnp.dot(t, wr2_ref[...],
                           preferred_element_type=jnp.float32) + br2_ref[...]


_tc_head = pl.pallas_call(
    _tc_head_body,
    out_shape=jax.ShapeDtypeStruct((1, 1), jnp.float32),
)


def kernel(x, edge_index, edge_attr, W_in, b_in, We, be, W1, b1, W2, b2,
           gamma, beta, Wr1, br1, Wr2, br2):
    src_g = edge_index[0]
    dst_g = edge_index[1]
    mask = dst_g < _NS
    m32 = mask.astype(jnp.int32)
    pos = jnp.where(mask, jnp.cumsum(m32) - 1,
                    _EPADH + jnp.cumsum(1 - m32) - 1)
    srcp = jnp.zeros((2 * _EPADH,), jnp.int32).at[pos].set(src_g)
    dstp = jnp.full((2 * _EPADH,), _NS, jnp.int32).at[pos].set(
        jnp.where(mask, dst_g, dst_g - _NS))
    eap = jnp.zeros((2 * _EPADH, 4), jnp.float32).at[pos].set(edge_attr)
    src2d = srcp.reshape(_ROWS, 128)
    dst2d = dstp.reshape(_ROWS, 128)

    ebias = [_tc_ebias(eap, We[l][None], be[l].reshape(1, _H))
             for l in range(_L)]

    hlo, hhi = _tc_in(x, W_in, b_in.reshape(1, _H))
    g = None
    for l in range(_L):
        eblo, ebhi = ebias[l]
        agg0 = _sc_layer_x0(hlo, hhi, src2d, dst2d, eblo, ebhi)
        agg1 = _sc_layer_x1(hlo, hhi, src2d, dst2d, eblo, ebhi)
        args = (hlo, hhi, agg0, agg0, agg1, agg1,
                W1[l], b1[l].reshape(1, -1), W2[l], b2[l].reshape(1, -1),
                gamma[l].reshape(1, -1), beta[l].reshape(1, -1))
        if l < _L - 1:
            hlo, hhi = _tc_up(*args)
        else:
            g = _tc_up_last(*args)
    out = _tc_head(g, Wr1, br1.reshape(1, -1), Wr2, br2.reshape(1, -1))
    return out.reshape(())


# speedup vs baseline: 2.0853x; 2.0853x over previous
import functools

import jax
import jax.numpy as jnp
from jax import lax
from jax.experimental import pallas as pl
from jax.experimental.pallas import tpu as pltpu
from jax.experimental.pallas import tpu_sc as plsc

_N = 50000
_E = 800000
_D_IN = 176
_H = 64
_HQ = 16
_L = 4

_EPAD = 819200
_ROWS = _EPAD // 128
_NSUB = 16
_RPT = _ROWS // _NSUB
_CROWS = 8
_CHUNK = _CROWS * 128
_NCHUNK = _RPT // _CROWS
_AGGR = 50176
_STRIPE = _AGGR // _NSUB
_ZROWS = 784
_ZREP = _STRIPE // _ZROWS

_BR = 2000


def _sc_layer_body(q, h_hbm, src_hbm, dst_hbm, eb0_hbm, eb1_hbm, out_hbm,
                   gbuf0, gbuf1, eb0, eb1, sv0, sv1, dv0, dv1, aggsp,
                   gsem0, gsem1, ssem0, ssem1):
    c = lax.axis_index("c")
    s = lax.axis_index("s")

    zv = jnp.zeros((16,), jnp.float32)

    def zrow(i, carry):
        gbuf0[i] = zv
        return carry

    lax.fori_loop(0, _ZROWS, zrow, 0)
    base = s * _STRIPE
    for t in range(_ZREP):
        pltpu.sync_copy(gbuf0.at[pl.ds(0, _ZROWS)],
                        aggsp.at[pl.ds(base + t * _ZROWS, _ZROWS)])
    plsc.subcore_barrier()

    row0 = s * _RPT
    goff = ((2 * q + c) * _N).astype(jnp.int32)

    set0 = (gbuf0, eb0, sv0, dv0, gsem0, ssem0)
    set1 = (gbuf1, eb1, sv1, dv1, gsem1, ssem1)

    def prefetch(g, bufs):
        gbuf, ebuf, sv, dv, gsem, _ = bufs
        rb = row0 + g * _CROWS
        pltpu.sync_copy(src_hbm.at[pl.ds(rb, _CROWS)], sv)
        pltpu.sync_copy(dst_hbm.at[pl.ds(rb, _CROWS)], dv)

        @pl.when(c == 0)
        def _():
            pltpu.sync_copy(eb0_hbm.at[pl.ds(rb * 128, _CHUNK)], ebuf)

        @pl.when(c == 1)
        def _():
            pltpu.sync_copy(eb1_hbm.at[pl.ds(rb * 128, _CHUNK)], ebuf)

        def adjrow(j, cr):
            for jj in range(8):
                sl = pl.ds(jj * 16, 16)
                sv[j, sl] = sv[j, sl] + goff
            return cr

        lax.fori_loop(0, _CROWS, adjrow, 0)
        for j in range(_CROWS):
            pltpu.async_copy(h_hbm.at[sv.at[j]],
                             gbuf.at[pl.ds(j * 128, 128)], gsem)

    def wait_n(sem, gbuf):
        for j in range(_CROWS):
            pltpu.make_async_copy(h_hbm.at[pl.ds(0, 128)],
                                  gbuf.at[pl.ds(0, 128)], sem).wait()

    def process(bufs):
        gbuf, ebuf, sv, dv, gsem, ssem = bufs
        wait_n(gsem, gbuf)

        def edge8(t, cr):
            e0 = t * 8
            for i in range(8):
                e = e0 + i
                gbuf[e] = jnp.maximum(gbuf[e] + ebuf[e], 0.0)
            return cr

        lax.fori_loop(0, _CHUNK // 8, edge8, 0)
        for j in range(_CROWS):
            pltpu.async_copy(gbuf.at[pl.ds(j * 128, 128)],
                             aggsp.at[dv.at[j]], ssem, add=True)

    def drain(bufs):
        gbuf, _, _, _, _, ssem = bufs
        wait_n(ssem, gbuf)

    prefetch(0, set0)
    prefetch(1, set1)
    process(set0)

    def pairbody(i, cr):
        t = 1 + 2 * i
        drain(set0)
        prefetch(t + 1, set0)
        process(set1)
        drain(set1)
        prefetch(t + 2, set1)
        process(set0)
        return cr

    lax.fori_loop(0, (_NCHUNK - 2) // 2, pairbody, 0)
    process(set1)
    drain(set0)
    drain(set1)
    plsc.subcore_barrier()

    obase = c * _AGGR + s * _STRIPE
    for t in range(_ZREP):
        pltpu.sync_copy(aggsp.at[pl.ds(base + t * _ZROWS, _ZROWS)],
                        gbuf0.at[pl.ds(0, _ZROWS)])
        pltpu.sync_copy(gbuf0.at[pl.ds(0, _ZROWS)],
                        out_hbm.at[pl.ds(obase + t * _ZROWS, _ZROWS)])


def _make_sc_layer(q):
    return pl.kernel(
        functools.partial(_sc_layer_body, q),
        out_type=jax.ShapeDtypeStruct((2 * _AGGR, _HQ), jnp.float32),
        mesh=plsc.VectorSubcoreMesh(core_axis_name="c", subcore_axis_name="s",
                                    num_cores=2, num_subcores=_NSUB),
        scratch_types=[
            pltpu.VMEM((_CHUNK, _HQ), jnp.float32),
            pltpu.VMEM((_CHUNK, _HQ), jnp.float32),
            pltpu.VMEM((_CHUNK, _HQ), jnp.float32),
            pltpu.VMEM((_CHUNK, _HQ), jnp.float32),
            pltpu.VMEM((_CROWS, 128), jnp.int32),
            pltpu.VMEM((_CROWS, 128), jnp.int32),
            pltpu.VMEM((_CROWS, 128), jnp.int32),
            pltpu.VMEM((_CROWS, 128), jnp.int32),
            pltpu.VMEM_SHARED((_AGGR, _HQ), jnp.float32),
            pltpu.SemaphoreType.DMA,
            pltpu.SemaphoreType.DMA,
            pltpu.SemaphoreType.DMA,
            pltpu.SemaphoreType.DMA,
        ],
        compiler_params=pltpu.CompilerParams(use_tc_tiling_on_sc=False),
    )


_sc_layer_q0 = _make_sc_layer(0)
_sc_layer_q1 = _make_sc_layer(1)


_BRE = 8192


def _tc_ebias_body(ea_ref, we_ref, be_ref, o0_ref, o1_ref, o2_ref, o3_ref):
    h = be_ref[...]
    for k in range(4):
        h = h + ea_ref[:, k:k + 1] * we_ref[0, k:k + 1, :]
    outs = (o0_ref, o1_ref, o2_ref, o3_ref)
    for t in range(4):
        outs[t][...] = h[:, t * _HQ:(t + 1) * _HQ]


_tc_ebias = pl.pallas_call(
    _tc_ebias_body,
    grid=(_EPAD // _BRE,),
    in_specs=[
        pl.BlockSpec((_BRE, 4), lambda i: (i, 0)),
        pl.BlockSpec((1, 4, _H), lambda i: (0, 0, 0)),
        pl.BlockSpec((1, _H), lambda i: (0, 0)),
    ],
    out_specs=[pl.BlockSpec((_BRE, _HQ), lambda i: (i, 0))] * 4,
    out_shape=[jax.ShapeDtypeStruct((_EPAD, _HQ), jnp.float32)] * 4,
)


def _tc_in_body(x_ref, w_ref, b_ref, out_ref):
    h = jnp.dot(x_ref[...], w_ref[...],
                preferred_element_type=jnp.float32) + b_ref[...]
    for t in range(4):
        out_ref[t] = h[:, t * _HQ:(t + 1) * _HQ]


_tc_in = pl.pallas_call(
    _tc_in_body,
    grid=(_N // _BR,),
    in_specs=[
        pl.BlockSpec((_BR, _D_IN), lambda i: (i, 0)),
        pl.BlockSpec((_D_IN, _H), lambda i: (0, 0)),
        pl.BlockSpec((1, _H), lambda i: (0, 0)),
    ],
    out_specs=pl.BlockSpec((4, _BR, _HQ), lambda i: (0, i, 0)),
    out_shape=jax.ShapeDtypeStruct((4, _N, _HQ), jnp.float32),
)


def _node_update(h_ref, agga_ref, aggb_ref, w1_ref, b1_ref, w2_ref, b2_ref,
                 gm_ref, bt_ref):
    z = jnp.concatenate(
        [h_ref[0] + agga_ref[0], h_ref[1] + agga_ref[1],
         h_ref[2] + aggb_ref[0], h_ref[3] + aggb_ref[1]], axis=1)
    t = jnp.maximum(jnp.dot(z, w1_ref[...],
                            preferred_element_type=jnp.float32)
                    + b1_ref[...], 0.0)
    z2 = jnp.dot(t, w2_ref[...],
                 preferred_element_type=jnp.float32) + b2_ref[...]
    mu = jnp.mean(z2, axis=1, keepdims=True)
    var = jnp.mean((z2 - mu) ** 2, axis=1, keepdims=True)
    zn = (z2 - mu) * lax.rsqrt(var + 1e-5)
    return jnp.maximum(zn * gm_ref[...] + bt_ref[...], 0.0)


def _tc_up_body(h_ref, agga_ref, aggb_ref, w1_ref, b1_ref, w2_ref, b2_ref,
                gm_ref, bt_ref, out_ref):
    hn = _node_update(h_ref, agga_ref, aggb_ref, w1_ref, b1_ref, w2_ref,
                      b2_ref, gm_ref, bt_ref)
    for t in range(4):
        out_ref[t] = hn[:, t * _HQ:(t + 1) * _HQ]


def _tc_up_last_body(h_ref, agga_ref, aggb_ref, w1_ref, b1_ref, w2_ref,
                     b2_ref, gm_ref, bt_ref, out_ref):
    hn = _node_update(h_ref, agga_ref, aggb_ref, w1_ref, b1_ref, w2_ref,
                      b2_ref, gm_ref, bt_ref)

    @pl.when(pl.program_id(0) == 0)
    def _():
        out_ref[...] = jnp.zeros_like(out_ref)

    out_ref[...] += jnp.sum(hn, axis=0, keepdims=True)


_up_in_specs = [
    pl.BlockSpec((4, _BR, _HQ), lambda i: (0, i, 0)),
    pl.BlockSpec((2, _BR, _HQ), lambda i: (0, i, 0)),
    pl.BlockSpec((2, _BR, _HQ), lambda i: (0, i, 0)),
    pl.BlockSpec((_H, 2 * _H), lambda i: (0, 0)),
    pl.BlockSpec((1, 2 * _H), lambda i: (0, 0)),
    pl.BlockSpec((2 * _H, _H), lambda i: (0, 0)),
    pl.BlockSpec((1, _H), lambda i: (0, 0)),
    pl.BlockSpec((1, _H), lambda i: (0, 0)),
    pl.BlockSpec((1, _H), lambda i: (0, 0)),
]

_tc_up = pl.pallas_call(
    _tc_up_body,
    grid=(_N // _BR,),
    in_specs=_up_in_specs,
    out_specs=pl.BlockSpec((4, _BR, _HQ), lambda i: (0, i, 0)),
    out_shape=jax.ShapeDtypeStruct((4, _N, _HQ), jnp.float32),
)

_tc_up_last = pl.pallas_call(
    _tc_up_last_body,
    grid=(_N // _BR,),
    in_specs=_up_in_specs,
    out_specs=pl.BlockSpec((1, _H), lambda i: (0, 0)),
    out_shape=jax.ShapeDtypeStruct((1, _H), jnp.float32),
)


def _tc_head_body(g_ref, wr1_ref, br1_ref, wr2_ref, br2_ref, out_ref):
    t = jnp.maximum(jnp.dot(g_ref[...], wr1_ref[...],
                            preferred_element_type=jnp.float32)
                    + br1_ref[...], 0.0)
    out_ref[...] = jnp.dot(t, wr2_ref[...],
                           preferred_element_type=jnp.float32) + br2_ref[...]


_tc_head = pl.pallas_call(
    _tc_head_body,
    out_shape=jax.ShapeDtypeStruct((1, 1), jnp.float32),
)


def kernel(x, edge_index, edge_attr, W_in, b_in, We, be, W1, b1, W2, b2,
           gamma, beta, Wr1, br1, Wr2, br2):
    pad = _EPAD - _E
    src2d = jnp.concatenate(
        [edge_index[0], jnp.zeros((pad,), jnp.int32)]).reshape(_ROWS, 128)
    dst2d = jnp.concatenate(
        [edge_index[1], jnp.full((pad,), _N, jnp.int32)]).reshape(_ROWS, 128)
    ea_pad = jnp.pad(edge_attr, ((0, pad), (0, 0)))

    ebias = [_tc_ebias(ea_pad, We[l][None], be[l].reshape(1, _H))
             for l in range(_L)]

    h4 = _tc_in(x, W_in, b_in.reshape(1, _H))
    g = None
    for l in range(_L):
        hflat = h4.reshape(4 * _N, _HQ)
        eb = ebias[l]
        agg_a = _sc_layer_q0(hflat, src2d, dst2d, eb[0], eb[1])
        agg_b = _sc_layer_q1(hflat, src2d, dst2d, eb[2], eb[3])
        args = (h4, agg_a.reshape(2, _AGGR, _HQ), agg_b.reshape(2, _AGGR, _HQ),
                W1[l], b1[l].reshape(1, -1), W2[l], b2[l].reshape(1, -1),
                gamma[l].reshape(1, -1), beta[l].reshape(1, -1))
        if l < _L - 1:
            h4 = _tc_up(*args)
        else:
            g = _tc_up_last(*args)
    out = _tc_head(g, Wr1, br1.reshape(1, -1), Wr2, br2.reshape(1, -1))
    return out.reshape(())
